# baseline (speedup 1.0000x reference)
import jax
import jax.numpy as jnp
from jax import lax
from jax.experimental import pallas as pl
from jax.experimental.pallas import tpu as pltpu
from jax.experimental.pallas import tpu_sc as plsc

N = 10000
E = 160000
D = 256
NC, NS, L = 2, 16, 16
NW = NC * NS
HALF_N = N // 2
HALF_P = 5120
NP = 2 * HALF_P
RPT = HALF_P // NS
EA = 160256
EPT_A = EA // NW
GA = EPT_A // L
SEG = 4000
NSEG = E // SEG
SBLK = 2000
NB = E // SBLK
CK = 16
ND = 4
CSLOT = SEG + 2 * L + 32
AGGR = RPT + 8

_mesh = lambda: plsc.VectorSubcoreMesh(core_axis_name="c", subcore_axis_name="s")


def _deg_body(dst_hbm, degp_hbm, dstv, degv):
    c = lax.axis_index("c")
    s = lax.axis_index("s")
    wid = s * NC + c

    def zero(i, carry):
        degv[pl.ds(i * L, L)] = jnp.zeros((L,), jnp.float32)
        return carry

    lax.fori_loop(0, NP // L, zero, 0)
    pltpu.sync_copy(dst_hbm.at[pl.ds(wid * EPT_A, EPT_A)], dstv)
    ones = jnp.ones((L,), jnp.float32)
    off = jnp.full((L,), HALF_P - HALF_N, jnp.int32)
    zoff = jnp.zeros((L,), jnp.int32)

    def body(g, carry):
        d = dstv[pl.ds(g * L, L)]
        idx = d + jnp.where(d >= HALF_N, off, zoff)
        plsc.addupdate_scatter(degv, [idx], ones)
        return carry

    lax.fori_loop(0, GA, body, 0)
    pltpu.sync_copy(degv.at[pl.ds(0, HALF_P)],
                    degp_hbm.at[pl.ds(wid * HALF_P, HALF_P)])
    pltpu.sync_copy(degv.at[pl.ds(HALF_P, HALF_P)],
                    degp_hbm.at[pl.ds((NW + wid) * HALF_P, HALF_P)])


def _make_deg():
    return pl.kernel(
        _deg_body,
        out_type=jax.ShapeDtypeStruct((NC * NW * HALF_P,), jnp.float32),
        mesh=_mesh(),
        scratch_types=[
            pltpu.VMEM((EPT_A,), jnp.int32),
            pltpu.VMEM((NP,), jnp.float32),
        ],
        compiler_params=pltpu.CompilerParams(needs_layout_passes=False),
    )


def _agg_body(xs_hbm, src_hbm, dst_hbm, agg_hbm,
              stgs0, stgd0, stgs1, stgd1, csrc, cdst,
              gbuf0, gbuf1, gbuf2, gbuf3, aggf,
              sem_a, sem_b, sem_g0, sem_g1, sem_g2, sem_g3):
    gbuf = [gbuf0, gbuf1, gbuf2, gbuf3]
    sem_g = [sem_g0, sem_g1, sem_g2, sem_g3]
    c = lax.axis_index("c")
    s = lax.axis_index("s")
    own = (c * NS + s) * RPT

    off = jnp.full((L,), HALF_P - HALF_N, jnp.int32)
    zoff = jnp.zeros((L,), jnp.int32)
    trash = jnp.full((L,), RPT, jnp.int32)
    zero16 = jnp.zeros((L,), jnp.int32)

    def stage_issue(g, stgs, stgd, sem):
        eoff = g * SBLK
        pltpu.async_copy(src_hbm.at[pl.ds(eoff, SBLK)], stgs, sem)
        pltpu.async_copy(dst_hbm.at[pl.ds(eoff, SBLK)], stgd, sem)

    def stage_wait(stgs, stgd, sem):
        pltpu.make_async_copy(src_hbm.at[pl.ds(0, SBLK)], stgs, sem).wait()
        pltpu.make_async_copy(dst_hbm.at[pl.ds(0, SBLK)], stgd, sem).wait()

    def scan_block(stgs, stgd, cnt):
        def grp(g, cnt2):
            sv = stgs[pl.ds(g * L, L)]
            d = stgd[pl.ds(g * L, L)]
            drow = d + jnp.where(d >= HALF_N, off, zoff)
            inr = (drow >= own) & (drow < own + RPT)
            srow = sv + jnp.where(sv >= HALF_N, off, zoff)
            plsc.store_compressed(csrc.at[pl.ds(cnt2, L)], srow, mask=inr)
            plsc.store_compressed(cdst.at[pl.ds(cnt2, L)], drow - own,
                                  mask=inr)
            pc = plsc.all_reduce_population_count(inr)
            return cnt2 + pc[0]

        return lax.fori_loop(0, SBLK // L, grp, cnt)

    def gather_issue(ci, k):
        pltpu.async_copy(xs_hbm.at[csrc.at[pl.ds(ci * CK, CK)]],
                         gbuf[k], sem_g[k])

    def gather_wait(k):
        pltpu.make_async_copy(xs_hbm.at[pl.ds(0, CK)], gbuf[k],
                              sem_g[k]).wait()

    def acc_chunk(ci, buf):
        for half in range(CK // L):
            dv = cdst[pl.ds(ci * CK + half * L, L)]
            for r in range(L):
                dl = dv[r]
                rr = half * L + r
                vals = [buf[rr, pl.ds(g2 * L, L)] for g2 in range(D // L)]
                for g2 in range(D // L):
                    plsc.addupdate(aggf.at[dl, pl.ds(g2 * L, L)], vals[g2])

    for j in range(RPT // 64):
        pltpu.sync_copy(xs_hbm.at[pl.ds(own + j * 64, 64)],
                        aggf.at[pl.ds(j * 64, 64)])

    stage_issue(0, stgs0, stgd0, sem_a)

    def segment(seg, carry):
        g0i = seg * (SEG // SBLK)
        stage_issue(g0i + 1, stgs1, stgd1, sem_b)
        stage_wait(stgs0, stgd0, sem_a)
        with jax.named_scope("sc_scan"):
            cnt = scan_block(stgs0, stgd0, jnp.int32(0))

        @pl.when(g0i + 2 < NB)
        def _():
            stage_issue(g0i + 2, stgs0, stgd0, sem_a)

        stage_wait(stgs1, stgd1, sem_b)
        with jax.named_scope("sc_scan"):
            cnt = scan_block(stgs1, stgd1, cnt)

        cdst[pl.ds(cnt, L)] = trash
        cdst[pl.ds(cnt + L, L)] = trash
        csrc[pl.ds(cnt, L)] = zero16
        csrc[pl.ds(cnt + L, L)] = zero16
        nch = lax.div(cnt + (CK - 1), jnp.int32(CK))

        for k in range(ND):
            @pl.when(k < nch)
            def _(k=k):
                gather_issue(jnp.int32(k), k)

        def quad(i, carry2):
            for k in range(ND):
                ck = i * ND + k

                @pl.when(ck < nch)
                def _(ck=ck, k=k):
                    with jax.named_scope("sc_gwait"):
                        gather_wait(k)
                    with jax.named_scope("sc_add"):
                        acc_chunk(ck, gbuf[k])

                    @pl.when(ck + ND < nch)
                    def _(ck=ck, k=k):
                        gather_issue(ck + ND, k)

            return carry2

        with jax.named_scope("sc_accum"):
            lax.fori_loop(0, lax.div(nch + (ND - 1), jnp.int32(ND)),
                          quad, 0)
        return carry

    lax.fori_loop(0, NSEG, segment, 0)

    for j in range(RPT // 64):
        pltpu.sync_copy(aggf.at[pl.ds(j * 64, 64)],
                        agg_hbm.at[pl.ds(own + j * 64, 64)])


def _make_agg():
    return pl.kernel(
        _agg_body,
        out_type=jax.ShapeDtypeStruct((NP, D), jnp.float32),
        mesh=_mesh(),
        scratch_types=[
            pltpu.VMEM((SBLK,), jnp.int32),
            pltpu.VMEM((SBLK,), jnp.int32),
            pltpu.VMEM((SBLK,), jnp.int32),
            pltpu.VMEM((SBLK,), jnp.int32),
            pltpu.VMEM((CSLOT,), jnp.int32),
            pltpu.VMEM((CSLOT,), jnp.int32),
            pltpu.VMEM((CK, D), jnp.float32),
            pltpu.VMEM((CK, D), jnp.float32),
            pltpu.VMEM((CK, D), jnp.float32),
            pltpu.VMEM((CK, D), jnp.float32),
            pltpu.VMEM((AGGR, D), jnp.float32),
            pltpu.SemaphoreType.DMA,
            pltpu.SemaphoreType.DMA,
            pltpu.SemaphoreType.DMA,
            pltpu.SemaphoreType.DMA,
            pltpu.SemaphoreType.DMA,
            pltpu.SemaphoreType.DMA,
        ],
        compiler_params=pltpu.CompilerParams(needs_layout_passes=False),
    )


def _scale_body(degp_ref, emb_ref, xs_ref, dis_ref):
    deg = jnp.sum(degp_ref[...], axis=1) + 1.0
    dis = lax.rsqrt(deg)
    dis_ref[...] = dis[..., None]
    xs_ref[...] = emb_ref[...] * dis[..., None]


def _make_scale():
    blk = HALF_P // 5
    return pl.pallas_call(
        _scale_body,
        grid=(NC, 5),
        in_specs=[
            pl.BlockSpec((1, NW, blk), lambda h, j: (h, 0, j)),
            pl.BlockSpec((1, blk, D), lambda h, j: (h, j, 0)),
        ],
        out_specs=[
            pl.BlockSpec((1, blk, D), lambda h, j: (h, j, 0)),
            pl.BlockSpec((1, blk, 1), lambda h, j: (h, j, 0)),
        ],
        out_shape=[
            jax.ShapeDtypeStruct((NC, HALF_P, D), jnp.float32),
            jax.ShapeDtypeStruct((NC, HALF_P, 1), jnp.float32),
        ],
    )


def _w_body(wg_ref, wd_ref, bg_ref, bd_ref, wc_ref, bc_ref):
    wd = wd_ref[...]
    wc_ref[...] = jnp.dot(wg_ref[...], wd, preferred_element_type=jnp.float32)
    bc_ref[...] = (
        jnp.dot(bg_ref[...], wd, preferred_element_type=jnp.float32)
        + bd_ref[...]
    )


def _make_wfold():
    return pl.pallas_call(
        _w_body,
        out_shape=[
            jax.ShapeDtypeStruct((D, D), jnp.float32),
            jax.ShapeDtypeStruct((1, D), jnp.float32),
        ],
    )


def _final_body(aggp_ref, dis_ref, wc_ref, bc_ref, out_ref):
    a = aggp_ref[0] * dis_ref[0]
    out_ref[0] = (
        jnp.dot(a, wc_ref[...], preferred_element_type=jnp.float32)
        + bc_ref[...]
    )


def _make_final():
    blk = HALF_P // 5
    return pl.pallas_call(
        _final_body,
        grid=(NC, 5),
        in_specs=[
            pl.BlockSpec((1, blk, D), lambda h, j: (h, j, 0)),
            pl.BlockSpec((1, blk, 1), lambda h, j: (h, j, 0)),
            pl.BlockSpec((D, D), lambda h, j: (0, 0)),
            pl.BlockSpec((1, D), lambda h, j: (0, 0)),
        ],
        out_specs=pl.BlockSpec((1, blk, D), lambda h, j: (h, j, 0)),
        out_shape=jax.ShapeDtypeStruct((NC, HALF_P, D), jnp.float32),
    )


def kernel(embedding, graph, W_gcn, b_gcn, W_dense, b_dense):
    src = graph[0]
    dst = graph[1]
    dst_a = jnp.concatenate([dst, jnp.full((EA - E,), N, jnp.int32)])
    pad = HALF_P - HALF_N
    emb_sp = jnp.stack([
        jnp.pad(embedding[:HALF_N], ((0, pad), (0, 0))),
        jnp.pad(embedding[HALF_N:], ((0, pad), (0, 0))),
    ])

    degp = _make_deg()(dst_a).reshape(NC, NW, HALF_P)
    xs_sp, dis_sp = _make_scale()(degp, emb_sp)
    wc, bc = _make_wfold()(W_gcn, W_dense,
                           b_gcn.reshape(1, D), b_dense.reshape(1, D))
    aggp = _make_agg()(xs_sp.reshape(NP, D), src, dst)
    out_sp = _make_final()(aggp.reshape(NC, HALF_P, D), dis_sp, wc, bc)
    return jnp.concatenate([out_sp[0, :HALF_N], out_sp[1, :HALF_N]], axis=0)


# speedup vs baseline: 1.0527x; 1.0527x over previous
import jax
import jax.numpy as jnp
from jax import lax
from jax.experimental import pallas as pl
from jax.experimental.pallas import tpu as pltpu
from jax.experimental.pallas import tpu_sc as plsc

N = 10000
E = 160000
D = 256
NC, NS, L = 2, 16, 16
NW = NC * NS
HALF_N = N // 2
HALF_P = 5120
NP = 2 * HALF_P
RPT = HALF_P // NS
EA = 160256
EPT_A = EA // NW
GA = EPT_A // L
SEG = 4000
NSEG = E // SEG
SBLK = 2000
NB = E // SBLK
CK = 16
ND = 2
CSLOT = SEG + 2 * L + 32
AGGR = RPT + 8

_mesh = lambda: plsc.VectorSubcoreMesh(core_axis_name="c", subcore_axis_name="s")


def _deg_body(dst_hbm, degp_hbm, dstv, degv):
    c = lax.axis_index("c")
    s = lax.axis_index("s")
    wid = s * NC + c

    def zero(i, carry):
        degv[pl.ds(i * L, L)] = jnp.zeros((L,), jnp.float32)
        return carry

    lax.fori_loop(0, NP // L, zero, 0)
    pltpu.sync_copy(dst_hbm.at[pl.ds(wid * EPT_A, EPT_A)], dstv)
    ones = jnp.ones((L,), jnp.float32)
    off = jnp.full((L,), HALF_P - HALF_N, jnp.int32)
    zoff = jnp.zeros((L,), jnp.int32)

    def body(g, carry):
        d = dstv[pl.ds(g * L, L)]
        idx = d + jnp.where(d >= HALF_N, off, zoff)
        plsc.addupdate_scatter(degv, [idx], ones)
        return carry

    lax.fori_loop(0, GA, body, 0)
    pltpu.sync_copy(degv.at[pl.ds(0, HALF_P)],
                    degp_hbm.at[pl.ds(wid * HALF_P, HALF_P)])
    pltpu.sync_copy(degv.at[pl.ds(HALF_P, HALF_P)],
                    degp_hbm.at[pl.ds((NW + wid) * HALF_P, HALF_P)])


def _make_deg():
    return pl.kernel(
        _deg_body,
        out_type=jax.ShapeDtypeStruct((NC * NW * HALF_P,), jnp.float32),
        mesh=_mesh(),
        scratch_types=[
            pltpu.VMEM((EPT_A,), jnp.int32),
            pltpu.VMEM((NP,), jnp.float32),
        ],
        compiler_params=pltpu.CompilerParams(needs_layout_passes=False),
    )


def _agg_body(xs_hbm, src_hbm, dst_hbm, agg_hbm,
              stgs0, stgd0, stgs1, stgd1, csrc, cdst,
              gbuf0, gbuf1, aggf,
              sem_a, sem_b, sem_g0, sem_g1):
    gbuf = [gbuf0, gbuf1]
    sem_g = [sem_g0, sem_g1]
    c = lax.axis_index("c")
    s = lax.axis_index("s")
    own = (c * NS + s) * RPT

    off = jnp.full((L,), HALF_P - HALF_N, jnp.int32)
    zoff = jnp.zeros((L,), jnp.int32)
    trash = jnp.full((L,), RPT, jnp.int32)
    zero16 = jnp.zeros((L,), jnp.int32)

    def stage_issue(g, stgs, stgd, sem):
        eoff = g * SBLK
        pltpu.async_copy(src_hbm.at[pl.ds(eoff, SBLK)], stgs, sem)
        pltpu.async_copy(dst_hbm.at[pl.ds(eoff, SBLK)], stgd, sem)

    def stage_wait(stgs, stgd, sem):
        pltpu.make_async_copy(src_hbm.at[pl.ds(0, SBLK)], stgs, sem).wait()
        pltpu.make_async_copy(dst_hbm.at[pl.ds(0, SBLK)], stgd, sem).wait()

    def scan_block(stgs, stgd, cnt):
        def grp(g, cnt2):
            sv = stgs[pl.ds(g * L, L)]
            d = stgd[pl.ds(g * L, L)]
            drow = d + jnp.where(d >= HALF_N, off, zoff)
            inr = (drow >= own) & (drow < own + RPT)
            srow = sv + jnp.where(sv >= HALF_N, off, zoff)
            plsc.store_compressed(csrc.at[pl.ds(cnt2, L)], srow, mask=inr)
            plsc.store_compressed(cdst.at[pl.ds(cnt2, L)], drow - own,
                                  mask=inr)
            pc = plsc.all_reduce_population_count(inr)
            return cnt2 + pc[0]

        return lax.fori_loop(0, SBLK // L, grp, cnt)

    def gather_issue(ci, k):
        pltpu.async_copy(xs_hbm.at[csrc.at[pl.ds(ci * CK, CK)]],
                         gbuf[k], sem_g[k])

    def gather_wait(k):
        pltpu.make_async_copy(xs_hbm.at[pl.ds(0, CK)], gbuf[k],
                              sem_g[k]).wait()

    def acc_chunk(ci, buf):
        for half in range(CK // L):
            dv = cdst[pl.ds(ci * CK + half * L, L)]
            for r in range(L):
                dl = dv[r]
                rr = half * L + r
                vals = [buf[rr, pl.ds(g2 * L, L)] for g2 in range(D // L)]
                for g2 in range(D // L):
                    plsc.addupdate(aggf.at[dl, pl.ds(g2 * L, L)], vals[g2])

    for j in range(RPT // 64):
        pltpu.sync_copy(xs_hbm.at[pl.ds(own + j * 64, 64)],
                        aggf.at[pl.ds(j * 64, 64)])

    stage_issue(0, stgs0, stgd0, sem_a)

    def segment(seg, carry):
        g0i = seg * (SEG // SBLK)
        stage_issue(g0i + 1, stgs1, stgd1, sem_b)
        stage_wait(stgs0, stgd0, sem_a)
        cnt = scan_block(stgs0, stgd0, jnp.int32(0))

        @pl.when(g0i + 2 < NB)
        def _():
            stage_issue(g0i + 2, stgs0, stgd0, sem_a)

        stage_wait(stgs1, stgd1, sem_b)
        cnt = scan_block(stgs1, stgd1, cnt)

        cdst[pl.ds(cnt, L)] = trash
        cdst[pl.ds(cnt + L, L)] = trash
        csrc[pl.ds(cnt, L)] = zero16
        csrc[pl.ds(cnt + L, L)] = zero16
        nch = lax.div(cnt + (CK - 1), jnp.int32(CK))

        for k in range(ND):
            @pl.when(k < nch)
            def _(k=k):
                gather_issue(jnp.int32(k), k)

        def quad(i, carry2):
            for k in range(ND):
                ck = i * ND + k

                @pl.when(ck < nch)
                def _(ck=ck, k=k):
                    gather_wait(k)
                    acc_chunk(ck, gbuf[k])

                    @pl.when(ck + ND < nch)
                    def _(ck=ck, k=k):
                        gather_issue(ck + ND, k)

            return carry2

        lax.fori_loop(0, lax.div(nch + (ND - 1), jnp.int32(ND)),
                      quad, 0)
        return carry

    lax.fori_loop(0, NSEG, segment, 0)

    for j in range(RPT // 64):
        pltpu.sync_copy(aggf.at[pl.ds(j * 64, 64)],
                        agg_hbm.at[pl.ds(own + j * 64, 64)])


def _make_agg():
    return pl.kernel(
        _agg_body,
        out_type=jax.ShapeDtypeStruct((NP, D), jnp.float32),
        mesh=_mesh(),
        scratch_types=[
            pltpu.VMEM((SBLK,), jnp.int32),
            pltpu.VMEM((SBLK,), jnp.int32),
            pltpu.VMEM((SBLK,), jnp.int32),
            pltpu.VMEM((SBLK,), jnp.int32),
            pltpu.VMEM((CSLOT,), jnp.int32),
            pltpu.VMEM((CSLOT,), jnp.int32),
            pltpu.VMEM((CK, D), jnp.float32),
            pltpu.VMEM((CK, D), jnp.float32),
            pltpu.VMEM((AGGR, D), jnp.float32),
            pltpu.SemaphoreType.DMA,
            pltpu.SemaphoreType.DMA,
            pltpu.SemaphoreType.DMA,
            pltpu.SemaphoreType.DMA,
        ],
        compiler_params=pltpu.CompilerParams(needs_layout_passes=False),
    )


def _scale_body(degp_ref, emb_ref, xs_ref, dis_ref):
    deg = jnp.sum(degp_ref[...], axis=1) + 1.0
    dis = lax.rsqrt(deg)
    dis_ref[...] = dis[..., None]
    xs_ref[...] = emb_ref[...] * dis[..., None]


def _make_scale():
    blk = HALF_P // 5
    return pl.pallas_call(
        _scale_body,
        grid=(NC, 5),
        in_specs=[
            pl.BlockSpec((1, NW, blk), lambda h, j: (h, 0, j)),
            pl.BlockSpec((1, blk, D), lambda h, j: (h, j, 0)),
        ],
        out_specs=[
            pl.BlockSpec((1, blk, D), lambda h, j: (h, j, 0)),
            pl.BlockSpec((1, blk, 1), lambda h, j: (h, j, 0)),
        ],
        out_shape=[
            jax.ShapeDtypeStruct((NC, HALF_P, D), jnp.float32),
            jax.ShapeDtypeStruct((NC, HALF_P, 1), jnp.float32),
        ],
    )


def _w_body(wg_ref, wd_ref, bg_ref, bd_ref, wc_ref, bc_ref):
    wd = wd_ref[...]
    wc_ref[...] = jnp.dot(wg_ref[...], wd, preferred_element_type=jnp.float32)
    bc_ref[...] = (
        jnp.dot(bg_ref[...], wd, preferred_element_type=jnp.float32)
        + bd_ref[...]
    )


def _make_wfold():
    return pl.pallas_call(
        _w_body,
        out_shape=[
            jax.ShapeDtypeStruct((D, D), jnp.float32),
            jax.ShapeDtypeStruct((1, D), jnp.float32),
        ],
    )


def _final_body(aggp_ref, dis_ref, wc_ref, bc_ref, out_ref):
    a = aggp_ref[0] * dis_ref[0]
    out_ref[0] = (
        jnp.dot(a, wc_ref[...], preferred_element_type=jnp.float32)
        + bc_ref[...]
    )


def _make_final():
    blk = HALF_P // 5
    return pl.pallas_call(
        _final_body,
        grid=(NC, 5),
        in_specs=[
            pl.BlockSpec((1, blk, D), lambda h, j: (h, j, 0)),
            pl.BlockSpec((1, blk, 1), lambda h, j: (h, j, 0)),
            pl.BlockSpec((D, D), lambda h, j: (0, 0)),
            pl.BlockSpec((1, D), lambda h, j: (0, 0)),
        ],
        out_specs=pl.BlockSpec((1, blk, D), lambda h, j: (h, j, 0)),
        out_shape=jax.ShapeDtypeStruct((NC, HALF_P, D), jnp.float32),
    )


def kernel(embedding, graph, W_gcn, b_gcn, W_dense, b_dense):
    src = graph[0]
    dst = graph[1]
    dst_a = jnp.concatenate([dst, jnp.full((EA - E,), N, jnp.int32)])
    pad = HALF_P - HALF_N
    emb_sp = jnp.stack([
        jnp.pad(embedding[:HALF_N], ((0, pad), (0, 0))),
        jnp.pad(embedding[HALF_N:], ((0, pad), (0, 0))),
    ])

    degp = _make_deg()(dst_a).reshape(NC, NW, HALF_P)
    xs_sp, dis_sp = _make_scale()(degp, emb_sp)
    wc, bc = _make_wfold()(W_gcn, W_dense,
                           b_gcn.reshape(1, D), b_dense.reshape(1, D))
    aggp = _make_agg()(xs_sp.reshape(NP, D), src, dst)
    out_sp = _make_final()(aggp.reshape(NC, HALF_P, D), dis_sp, wc, bc)
    return jnp.concatenate([out_sp[0, :HALF_N], out_sp[1, :HALF_N]], axis=0)
